# baseline (speedup 1.0000x reference)
import functools

import jax
import jax.numpy as jnp
from jax import lax
from jax.experimental import pallas as pl
from jax.experimental.pallas import tpu as pltpu
from jax.experimental.pallas import tpu_sc as plsc

B = 16384
D = 32
NC, NS, L = 2, 16, 16
NW = NC * NS
BPW = B // NW
CHUNK = 128
NCHUNK = BPW // CHUNK
NG = BPW // L


def _sc_stage(user_embedding, item_embedding, u, i, j):
    mesh = plsc.VectorSubcoreMesh(core_axis_name="c", subcore_axis_name="s")

    @functools.partial(
        pl.kernel,
        mesh=mesh,
        compiler_params=pltpu.CompilerParams(use_tc_tiling_on_sc=False),
        out_type=[
            jax.ShapeDtypeStruct((B * L,), jnp.float32),
            jax.ShapeDtypeStruct((NW, L), jnp.float32),
        ],
        scratch_types=[
            pltpu.VMEM((BPW,), jnp.int32),
            pltpu.VMEM((BPW,), jnp.int32),
            pltpu.VMEM((BPW,), jnp.int32),
            pltpu.VMEM((BPW, D), jnp.float32),
            pltpu.VMEM((BPW, D), jnp.float32),
            pltpu.VMEM((BPW, D), jnp.float32),
            pltpu.VMEM((BPW * L,), jnp.float32),
            pltpu.VMEM((L,), jnp.float32),
            pltpu.SemaphoreType.DMA,
        ],
    )
    def k(user_hbm, item_hbm, u_hbm, i_hbm, j_hbm, h_hbm, reg_hbm,
          u_idx, i_idx, j_idx, u_rows, p_rows, n_rows, h_v, regv, sem):
        wid = lax.axis_index("s") * NC + lax.axis_index("c")
        base = wid * BPW
        pltpu.sync_copy(u_hbm.at[pl.ds(base, BPW)], u_idx)
        pltpu.sync_copy(i_hbm.at[pl.ds(base, BPW)], i_idx)
        pltpu.sync_copy(j_hbm.at[pl.ds(base, BPW)], j_idx)
        copies = []
        for c in range(NCHUNK):
            sl = pl.ds(c * CHUNK, CHUNK)
            copies.append(
                pltpu.async_copy(user_hbm.at[u_idx.at[sl]], u_rows.at[sl], sem))
            copies.append(
                pltpu.async_copy(item_hbm.at[i_idx.at[sl]], p_rows.at[sl], sem))
            copies.append(
                pltpu.async_copy(item_hbm.at[j_idx.at[sl]], n_rows.at[sl], sem))
        for cp in copies:
            cp.wait()

        def group(g, reg_acc):
            rb = g * L
            for r in range(L):
                row = rb + r
                u0 = u_rows[row, pl.ds(0, L)]
                u1 = u_rows[row, pl.ds(L, L)]
                p0 = p_rows[row, pl.ds(0, L)]
                p1 = p_rows[row, pl.ds(L, L)]
                n0 = n_rows[row, pl.ds(0, L)]
                n1 = n_rows[row, pl.ds(L, L)]
                h_v[pl.ds(row * L, L)] = u0 * (n0 - p0) + u1 * (n1 - p1)
                reg_acc = (reg_acc + u0 * u0 + u1 * u1 + p0 * p0
                           + p1 * p1 + n0 * n0 + n1 * n1)
            return reg_acc

        reg = lax.fori_loop(0, NG, group, jnp.zeros((L,), jnp.float32))
        regv[...] = reg
        pltpu.sync_copy(h_v, h_hbm.at[pl.ds(base * L, BPW * L)])
        pltpu.sync_copy(regv, reg_hbm.at[wid])

    return k(user_embedding, item_embedding, u, i, j)


def _tc_reduce(h, reg_partials):
    def body(h_ref, r_ref, bpr_ref, reg_ref):
        x = h_ref[...]
        col = lax.broadcasted_iota(jnp.int32, (128, 8), 0)
        grp = lax.broadcasted_iota(jnp.int32, (128, 8), 1)
        sel = jnp.where(col // L == grp, 1.0, 0.0).astype(jnp.float32)
        dm = jax.lax.dot_general(x, sel, (((1,), (0,)), ((), ())),
                                 preferred_element_type=jnp.float32)
        sp = jnp.maximum(dm, 0.0) + jnp.log(1.0 + jnp.exp(-jnp.abs(dm)))
        bpr_ref[...] = jnp.full((8, 128), jnp.sum(sp), jnp.float32)
        reg_ref[...] = jnp.full((8, 128), jnp.sum(r_ref[...]) * (1.0 / B),
                                jnp.float32)

    bpr, reg = pl.pallas_call(
        body,
        out_shape=[jax.ShapeDtypeStruct((8, 128), jnp.float32),
                   jax.ShapeDtypeStruct((8, 128), jnp.float32)],
    )(h.reshape(B * L // 128, 128), reg_partials.reshape(4, 128))
    return bpr[0, 0], reg[0, 0]


def kernel(user_embedding, item_embedding, u, i, j):
    u = u.astype(jnp.int32)
    i = i.astype(jnp.int32)
    j = j.astype(jnp.int32)
    h, reg_partials = _sc_stage(user_embedding, item_embedding, u, i, j)
    return _tc_reduce(h, reg_partials)
